# baseline (speedup 1.0000x reference)
import functools

import jax
import jax.numpy as jnp
from jax import lax
from jax.experimental import pallas as pl
from jax.experimental.pallas import tpu as pltpu
from jax.experimental.pallas import tpu_sc as plsc

B = 4096
D = 784
NUM_CLASSES = 10

NC = 2
NS = 16
NW = NC * NS
CHUNK = 64
NPASS = B // (NW * CHUNK)
OCN = D // 16

_LAYERS = ((0, 2, 0, True), (2, 4, 1, True), (6, 8, 2, False))
_KTOT = 14


def _gelu_exact(v):
    z = v * 0.7071067811865476
    a = jnp.abs(z)
    t = 1.0 / (1.0 + 0.3275911 * a)
    poly = t * (0.254829592 + t * (-0.284496736 + t * (1.421413741
             + t * (-1.453152027 + t * 1.061405429))))
    erf_a = 1.0 - poly * jnp.exp(-(a * a))
    erf_z = jnp.where(z < 0.0, -erf_a, erf_a)
    return 0.5 * v * (1.0 + erf_z)


def _gelu_fast(v):
    t = jnp.minimum(v * v, 90.0)
    u = -0.0007098086084286619 * t + 0.07405305138626019
    u = u * t + 1.5949698227920912
    e = jnp.exp(-(u * v))
    return v / (1.0 + e)


def _sc_layer(src, dst, idxs, ws, bsv, k0, kk, brow, do_gelu):
    @plsc.parallel_loop(0, OCN)
    def oc_body(oc):
        col = oc * 16
        bias = bsv[pl.ds(brow * D + col, 16)]
        taps = [(idxs[pl.ds((k0 + k) * D + col, 16)],
                 ws[pl.ds((k0 + k) * D + col, 16)]) for k in range(kk)]

        @plsc.parallel_loop(0, CHUNK, unroll=6)
        def b_body(b):
            boff = b * D
            acc = bias
            for rvec, wvec in taps:
                vals = plsc.load_gather(src, [rvec + boff])
                acc = acc + vals * wvec
            if do_gelu:
                acc = _gelu_fast(acc)
            dst[pl.ds(boff + col, 16)] = acc


def _make_sc_net():
    mesh = plsc.VectorSubcoreMesh(core_axis_name="c", subcore_axis_name="s",
                                  num_cores=NC, num_subcores=NS)

    @functools.partial(
        pl.kernel,
        out_type=jax.ShapeDtypeStruct((B, D), jnp.float32),
        mesh=mesh,
        compiler_params=pltpu.CompilerParams(
            needs_layout_passes=False, use_tc_tiling_on_sc=False),
        scratch_types=[
            pltpu.VMEM((CHUNK * D,), jnp.float32),
            pltpu.VMEM((CHUNK * D,), jnp.float32),
            pltpu.VMEM((_KTOT * D,), jnp.int32),
            pltpu.VMEM((_KTOT * D,), jnp.float32),
            pltpu.VMEM((3 * D,), jnp.float32),
            pltpu.SemaphoreType.DMA,
        ],
    )
    def sc_net(x_hbm, idx_hbm, w_hbm, b_hbm, out_hbm, xs, hs, idxs, ws, bsv, sem):
        wid = lax.axis_index("s") * NC + lax.axis_index("c")
        pltpu.sync_copy(idx_hbm, idxs)
        pltpu.sync_copy(w_hbm, ws)
        pltpu.sync_copy(b_hbm, bsv)
        for p in range(NPASS):
            row0 = (wid * NPASS + p) * CHUNK
            loads = [pltpu.async_copy(x_hbm.at[row0 + b],
                                      xs.at[pl.ds(b * D, D)], sem)
                     for b in range(CHUNK)]
            for cp in loads:
                cp.wait()
            _sc_layer(xs, hs, idxs, ws, bsv, *_LAYERS[0])
            _sc_layer(hs, xs, idxs, ws, bsv, *_LAYERS[1])
            _sc_layer(xs, hs, idxs, ws, bsv, *_LAYERS[2])
            stores = [pltpu.async_copy(hs.at[pl.ds(b * D, D)],
                                       out_hbm.at[row0 + b], sem)
                      for b in range(CHUNK)]
            for cp in stores:
                cp.wait()

    return sc_net


@functools.cache
def _sc_net_cached():
    return _make_sc_net()


def _tc_head_body(h_ref, w_ref, b_ref, o_ref):
    g = _gelu_fast(h_ref[...])
    logits = jnp.dot(g, w_ref[...], preferred_element_type=jnp.float32,
                     precision=lax.Precision.HIGHEST) + b_ref[...]
    m = jnp.max(logits, axis=-1, keepdims=True)
    e = jnp.exp(logits - m)
    o_ref[...] = e / jnp.sum(e, axis=-1, keepdims=True)


def _tc_head(h3, fcw_t, fc_b2):
    blk = 1024
    return pl.pallas_call(
        _tc_head_body,
        grid=(B // blk,),
        in_specs=[
            pl.BlockSpec((blk, D), lambda i: (i, 0)),
            pl.BlockSpec((D, NUM_CLASSES), lambda i: (0, 0)),
            pl.BlockSpec((1, NUM_CLASSES), lambda i: (0, 0)),
        ],
        out_specs=pl.BlockSpec((blk, NUM_CLASSES), lambda i: (i, 0)),
        out_shape=jax.ShapeDtypeStruct((B, NUM_CLASSES), jnp.float32),
    )(h3, fcw_t, fc_b2)


def kernel(x, idx1, w1, b1, idx2, w2, b2, idx3, w3, b3, fc_w, fc_b):
    idx_all = jnp.concatenate(
        [idx1.T.astype(jnp.int32), idx2.T.astype(jnp.int32),
         idx3.T.astype(jnp.int32)], axis=0).reshape(-1)
    w_all = jnp.concatenate([w1.T, w2.T, w3.T], axis=0).reshape(-1)
    b_all = jnp.concatenate([b1, b2, b3], axis=0)

    h3 = _sc_net_cached()(x, idx_all, w_all, b_all)
    return _tc_head(h3, fc_w.T, fc_b.reshape(1, NUM_CLASSES))


# speedup vs baseline: 1.2107x; 1.2107x over previous
import functools

import jax
import jax.numpy as jnp
from jax import lax
from jax.experimental import pallas as pl
from jax.experimental.pallas import tpu as pltpu
from jax.experimental.pallas import tpu_sc as plsc

B = 4096
D = 784
NUM_CLASSES = 10

NC = 2
NS = 16
NW = NC * NS
CHUNK = 64
NPASS = B // (NW * CHUNK)
OCN = D // 16

_LAYERS = ((0, 2, 0, True), (2, 4, 1, True), (6, 8, 2, False))
_KTOT = 14


def _gelu_exact(v):
    z = v * 0.7071067811865476
    a = jnp.abs(z)
    t = 1.0 / (1.0 + 0.3275911 * a)
    poly = t * (0.254829592 + t * (-0.284496736 + t * (1.421413741
             + t * (-1.453152027 + t * 1.061405429))))
    erf_a = 1.0 - poly * jnp.exp(-(a * a))
    erf_z = jnp.where(z < 0.0, -erf_a, erf_a)
    return 0.5 * v * (1.0 + erf_z)


def _gelu_fast(v):
    t = jnp.minimum(v * v, 90.0)
    u = -0.0007098086084286619 * t + 0.07405305138626019
    u = u * t + 1.5949698227920912
    e = jnp.exp(-(u * v))
    return v / (1.0 + e)


def _sc_layer(src, dst, idxs, ws, bsv, k0, kk, brow, do_gelu):
    @plsc.parallel_loop(0, OCN)
    def oc_body(oc):
        col = oc * 16
        bias = bsv[pl.ds(brow * D + col, 16)]
        taps = [(idxs[pl.ds((k0 + k) * D + col, 16)],
                 ws[pl.ds((k0 + k) * D + col, 16)]) for k in range(kk)]

        @plsc.parallel_loop(0, CHUNK, unroll=4)
        def b_body(b):
            boff = b * D
            acc = bias
            for rvec, wvec in taps:
                vals = plsc.load_gather(src, [rvec + boff])
                acc = acc + vals * wvec
            if do_gelu:
                acc = _gelu_fast(acc)
            dst[pl.ds(boff + col, 16)] = acc


def _make_sc_net():
    mesh = plsc.VectorSubcoreMesh(core_axis_name="c", subcore_axis_name="s",
                                  num_cores=NC, num_subcores=NS)

    @functools.partial(
        pl.kernel,
        out_type=jax.ShapeDtypeStruct((B, D), jnp.float32),
        mesh=mesh,
        compiler_params=pltpu.CompilerParams(
            needs_layout_passes=False, use_tc_tiling_on_sc=False),
        scratch_types=[
            pltpu.VMEM((CHUNK * D,), jnp.float32),
            pltpu.VMEM((CHUNK * D,), jnp.float32),
            pltpu.VMEM((_KTOT * D,), jnp.int32),
            pltpu.VMEM((_KTOT * D,), jnp.float32),
            pltpu.VMEM((3 * D,), jnp.float32),
            pltpu.SemaphoreType.DMA,
            pltpu.SemaphoreType.DMA,
        ],
    )
    def sc_net(x_hbm, idx_hbm, w_hbm, b_hbm, out_hbm,
               xs, hs, idxs, ws, bsv, sem, sem2):
        wid = lax.axis_index("s") * NC + lax.axis_index("c")
        pltpu.sync_copy(idx_hbm, idxs)
        pltpu.sync_copy(w_hbm, ws)
        pltpu.sync_copy(b_hbm, bsv)
        def fire_loads(p):
            row0 = (wid * NPASS + p) * CHUNK
            return [pltpu.async_copy(x_hbm.at[row0 + b],
                                     xs.at[pl.ds(b * D, D)], sem)
                    for b in range(CHUNK)]

        stores = []
        loads = fire_loads(0)
        for p in range(NPASS):
            for cp in loads:
                cp.wait()
            _sc_layer(xs, hs, idxs, ws, bsv, *_LAYERS[0])
            _sc_layer(hs, xs, idxs, ws, bsv, *_LAYERS[1])
            _sc_layer(xs, hs, idxs, ws, bsv, *_LAYERS[2])
            row0 = (wid * NPASS + p) * CHUNK
            stores = [pltpu.async_copy(hs.at[pl.ds(b * D, D)],
                                       out_hbm.at[row0 + b], sem2)
                      for b in range(CHUNK)]
            if p + 1 < NPASS:
                loads = fire_loads(p + 1)
            for cp in stores:
                cp.wait()

    return sc_net


@functools.cache
def _sc_net_cached():
    return _make_sc_net()


def _tc_head_body(h_ref, w_ref, b_ref, o_ref):
    g = _gelu_fast(h_ref[...])
    logits = jnp.dot(g, w_ref[...], preferred_element_type=jnp.float32,
                     precision=lax.Precision.HIGHEST) + b_ref[...]
    m = jnp.max(logits, axis=-1, keepdims=True)
    e = jnp.exp(logits - m)
    o_ref[...] = e / jnp.sum(e, axis=-1, keepdims=True)


def _tc_head(h3, fcw_t, fc_b2):
    blk = 1024
    return pl.pallas_call(
        _tc_head_body,
        grid=(B // blk,),
        in_specs=[
            pl.BlockSpec((blk, D), lambda i: (i, 0)),
            pl.BlockSpec((D, NUM_CLASSES), lambda i: (0, 0)),
            pl.BlockSpec((1, NUM_CLASSES), lambda i: (0, 0)),
        ],
        out_specs=pl.BlockSpec((blk, NUM_CLASSES), lambda i: (i, 0)),
        out_shape=jax.ShapeDtypeStruct((B, NUM_CLASSES), jnp.float32),
    )(h3, fcw_t, fc_b2)


def kernel(x, idx1, w1, b1, idx2, w2, b2, idx3, w3, b3, fc_w, fc_b):
    idx_all = jnp.concatenate(
        [idx1.T.astype(jnp.int32), idx2.T.astype(jnp.int32),
         idx3.T.astype(jnp.int32)], axis=0).reshape(-1)
    w_all = jnp.concatenate([w1.T, w2.T, w3.T], axis=0).reshape(-1)
    b_all = jnp.concatenate([b1, b2, b3], axis=0)

    h3 = _sc_net_cached()(x, idx_all, w_all, b_all)
    return _tc_head(h3, fc_w.T, fc_b.reshape(1, NUM_CLASSES))


# speedup vs baseline: 1.8905x; 1.5615x over previous
import functools

import jax
import jax.numpy as jnp
from jax import lax
from jax.experimental import pallas as pl
from jax.experimental.pallas import tpu as pltpu
from jax.experimental.pallas import tpu_sc as plsc

B = 4096
D = 784
NUM_CLASSES = 10

B_SC = 2048
B_TC = B - B_SC

NC = 2
NS = 16
NW = NC * NS
CHUNK = 64
NPASS = B_SC // (NW * CHUNK)
OCN = D // 16

_LAYERS = ((0, 2, 0, True), (2, 4, 1, True), (6, 8, 2, False))
_KTOT = 14


def _gelu_exact(v):
    z = v * 0.7071067811865476
    a = jnp.abs(z)
    t = 1.0 / (1.0 + 0.3275911 * a)
    poly = t * (0.254829592 + t * (-0.284496736 + t * (1.421413741
             + t * (-1.453152027 + t * 1.061405429))))
    erf_a = 1.0 - poly * jnp.exp(-(a * a))
    erf_z = jnp.where(z < 0.0, -erf_a, erf_a)
    return 0.5 * v * (1.0 + erf_z)


def _gelu_fast(v):
    t = jnp.minimum(v * v, 90.0)
    u = -0.0007098086084286619 * t + 0.07405305138626019
    u = u * t + 1.5949698227920912
    e = jnp.exp(-(u * v))
    return v / (1.0 + e)


def _sc_layer(src, dst, idxs, ws, bsv, k0, kk, brow, do_gelu):
    @plsc.parallel_loop(0, OCN)
    def oc_body(oc):
        col = oc * 16
        bias = bsv[pl.ds(brow * D + col, 16)]
        taps = [(idxs[pl.ds((k0 + k) * D + col, 16)],
                 ws[pl.ds((k0 + k) * D + col, 16)]) for k in range(kk)]

        @plsc.parallel_loop(0, CHUNK, unroll=4)
        def b_body(b):
            boff = b * D
            acc = bias
            for rvec, wvec in taps:
                vals = plsc.load_gather(src, [rvec + boff])
                acc = acc + vals * wvec
            if do_gelu:
                acc = _gelu_fast(acc)
            dst[pl.ds(boff + col, 16)] = acc


def _make_sc_net():
    mesh = plsc.VectorSubcoreMesh(core_axis_name="c", subcore_axis_name="s",
                                  num_cores=NC, num_subcores=NS)

    @functools.partial(
        pl.kernel,
        out_type=jax.ShapeDtypeStruct((B_SC, D), jnp.float32),
        mesh=mesh,
        compiler_params=pltpu.CompilerParams(
            needs_layout_passes=False, use_tc_tiling_on_sc=False),
        scratch_types=[
            pltpu.VMEM((CHUNK * D,), jnp.float32),
            pltpu.VMEM((CHUNK * D,), jnp.float32),
            pltpu.VMEM((_KTOT * D,), jnp.int32),
            pltpu.VMEM((_KTOT * D,), jnp.float32),
            pltpu.VMEM((3 * D,), jnp.float32),
            pltpu.SemaphoreType.DMA,
            pltpu.SemaphoreType.DMA,
        ],
    )
    def sc_net(x_hbm, idx_hbm, w_hbm, b_hbm, out_hbm,
               xs, hs, idxs, ws, bsv, sem, sem2):
        wid = lax.axis_index("s") * NC + lax.axis_index("c")
        pltpu.sync_copy(idx_hbm, idxs)
        pltpu.sync_copy(w_hbm, ws)
        pltpu.sync_copy(b_hbm, bsv)
        def fire_loads(p):
            row0 = (wid * NPASS + p) * CHUNK
            return [pltpu.async_copy(x_hbm.at[row0 + b],
                                     xs.at[pl.ds(b * D, D)], sem)
                    for b in range(CHUNK)]

        stores = []
        loads = fire_loads(0)
        for p in range(NPASS):
            for cp in loads:
                cp.wait()
            _sc_layer(xs, hs, idxs, ws, bsv, *_LAYERS[0])
            _sc_layer(hs, xs, idxs, ws, bsv, *_LAYERS[1])
            _sc_layer(xs, hs, idxs, ws, bsv, *_LAYERS[2])
            row0 = (wid * NPASS + p) * CHUNK
            stores = [pltpu.async_copy(hs.at[pl.ds(b * D, D)],
                                       out_hbm.at[row0 + b], sem2)
                      for b in range(CHUNK)]
            if p + 1 < NPASS:
                loads = fire_loads(p + 1)
            for cp in stores:
                cp.wait()

    return sc_net


@functools.cache
def _sc_net_cached():
    return _make_sc_net()


def _tc_head_body(h_ref, w_ref, b_ref, o_ref):
    g = _gelu_fast(h_ref[...])
    logits = jnp.dot(g, w_ref[...], preferred_element_type=jnp.float32,
                     precision=lax.Precision.HIGHEST) + b_ref[...]
    m = jnp.max(logits, axis=-1, keepdims=True)
    e = jnp.exp(logits - m)
    o_ref[...] = e / jnp.sum(e, axis=-1, keepdims=True)


def _tc_head(h3, fcw_t, fc_b2):
    blk = 1024
    nrows = h3.shape[0]
    return pl.pallas_call(
        _tc_head_body,
        grid=(nrows // blk,),
        in_specs=[
            pl.BlockSpec((blk, D), lambda i: (i, 0)),
            pl.BlockSpec((D, NUM_CLASSES), lambda i: (0, 0)),
            pl.BlockSpec((1, NUM_CLASSES), lambda i: (0, 0)),
        ],
        out_specs=pl.BlockSpec((blk, NUM_CLASSES), lambda i: (i, 0)),
        out_shape=jax.ShapeDtypeStruct((nrows, NUM_CLASSES), jnp.float32),
    )(h3, fcw_t, fc_b2)


def _densify_t(idx_ref, w_ref, kk):
    ii = lax.broadcasted_iota(jnp.int32, (D, D), 0)
    wt = jnp.zeros((D, D), jnp.float32)
    for k in range(kk):
        idx_k = idx_ref[...][:, k][None, :]
        w_k = w_ref[...][:, k][None, :]
        wt = wt + jnp.where(ii == idx_k, w_k, 0.0)
    return wt


def _tc_chain_body(x_ref, i1, w1r, b1r, i2, w2r, b2r, i3, w3r, b3r,
                   fw_ref, fb_ref, o_ref):
    h = x_ref[...]
    for iref, wref, bref, kk, g in ((i1, w1r, b1r, 2, True),
                                    (i2, w2r, b2r, 4, True),
                                    (i3, w3r, b3r, 8, False)):
        wt = _densify_t(iref, wref, kk)
        h = jnp.dot(h, wt, preferred_element_type=jnp.float32) + bref[...]
        if g:
            h = _gelu_fast(h)
    g3 = _gelu_fast(h)
    logits = jnp.dot(g3, fw_ref[...], preferred_element_type=jnp.float32) \
        + fb_ref[...]
    m = jnp.max(logits, axis=-1, keepdims=True)
    e = jnp.exp(logits - m)
    o_ref[...] = e / jnp.sum(e, axis=-1, keepdims=True)


def _tc_chain(x_tc, sparse_params, fcw_t, fc_b2):
    return pl.pallas_call(
        _tc_chain_body,
        out_shape=jax.ShapeDtypeStruct((B_TC, NUM_CLASSES), jnp.float32),
    )(x_tc, *sparse_params, fcw_t, fc_b2)


def kernel(x, idx1, w1, b1, idx2, w2, b2, idx3, w3, b3, fc_w, fc_b):
    idx_all = jnp.concatenate(
        [idx1.T.astype(jnp.int32), idx2.T.astype(jnp.int32),
         idx3.T.astype(jnp.int32)], axis=0).reshape(-1)
    w_all = jnp.concatenate([w1.T, w2.T, w3.T], axis=0).reshape(-1)
    b_all = jnp.concatenate([b1, b2, b3], axis=0)

    fcw_t = fc_w.T
    fc_b2 = fc_b.reshape(1, NUM_CLASSES)

    sparse_params = (
        idx1.astype(jnp.int32), w1, b1.reshape(1, D),
        idx2.astype(jnp.int32), w2, b2.reshape(1, D),
        idx3.astype(jnp.int32), w3, b3.reshape(1, D),
    )
    probs_tc = _tc_chain(x[B_SC:], sparse_params, fcw_t, fc_b2)
    h3 = _sc_net_cached()(x[:B_SC], idx_all, w_all, b_all)
    probs_sc = _tc_head(h3, fcw_t, fc_b2)
    return jnp.concatenate([probs_sc, probs_tc], axis=0)


# speedup vs baseline: 2.3959x; 1.2673x over previous
import functools

import jax
import jax.numpy as jnp
from jax import lax
from jax.experimental import pallas as pl
from jax.experimental.pallas import tpu as pltpu
from jax.experimental.pallas import tpu_sc as plsc

B = 4096
D = 784
NUM_CLASSES = 10

B_SC = 1024
B_TC = B - B_SC

NC = 2
NS = 16
NW = NC * NS
CHUNK = 32
NPASS = B_SC // (NW * CHUNK)
OCN = D // 16

_LAYERS = ((0, 2, 0, True), (2, 4, 1, True), (6, 8, 2, False))
_KTOT = 14


def _gelu_exact(v):
    z = v * 0.7071067811865476
    a = jnp.abs(z)
    t = 1.0 / (1.0 + 0.3275911 * a)
    poly = t * (0.254829592 + t * (-0.284496736 + t * (1.421413741
             + t * (-1.453152027 + t * 1.061405429))))
    erf_a = 1.0 - poly * jnp.exp(-(a * a))
    erf_z = jnp.where(z < 0.0, -erf_a, erf_a)
    return 0.5 * v * (1.0 + erf_z)


def _gelu_fast(v):
    t = jnp.minimum(v * v, 90.0)
    u = -0.0007098086084286619 * t + 0.07405305138626019
    u = u * t + 1.5949698227920912
    e = jnp.exp(-(u * v))
    return v / (1.0 + e)


def _sc_layer(src, dst, idxs, ws, bsv, k0, kk, brow, do_gelu):
    @plsc.parallel_loop(0, OCN)
    def oc_body(oc):
        col = oc * 16
        bias = bsv[pl.ds(brow * D + col, 16)]
        taps = [(idxs[pl.ds((k0 + k) * D + col, 16)],
                 ws[pl.ds((k0 + k) * D + col, 16)]) for k in range(kk)]

        @plsc.parallel_loop(0, CHUNK, unroll=4)
        def b_body(b):
            boff = b * D
            acc = bias
            for rvec, wvec in taps:
                vals = plsc.load_gather(src, [rvec + boff])
                acc = acc + vals * wvec
            if do_gelu:
                acc = _gelu_fast(acc)
            dst[pl.ds(boff + col, 16)] = acc


def _make_sc_net():
    mesh = plsc.VectorSubcoreMesh(core_axis_name="c", subcore_axis_name="s",
                                  num_cores=NC, num_subcores=NS)

    @functools.partial(
        pl.kernel,
        out_type=jax.ShapeDtypeStruct((B_SC, D), jnp.float32),
        mesh=mesh,
        compiler_params=pltpu.CompilerParams(
            needs_layout_passes=False, use_tc_tiling_on_sc=False),
        scratch_types=[
            pltpu.VMEM((CHUNK * D,), jnp.float32),
            pltpu.VMEM((CHUNK * D,), jnp.float32),
            pltpu.VMEM((_KTOT * D,), jnp.int32),
            pltpu.VMEM((_KTOT * D,), jnp.float32),
            pltpu.VMEM((3 * D,), jnp.float32),
            pltpu.SemaphoreType.DMA,
            pltpu.SemaphoreType.DMA,
        ],
    )
    def sc_net(x_hbm, idx_hbm, w_hbm, b_hbm, out_hbm,
               xs, hs, idxs, ws, bsv, sem, sem2):
        wid = lax.axis_index("s") * NC + lax.axis_index("c")
        pltpu.sync_copy(idx_hbm, idxs)
        pltpu.sync_copy(w_hbm, ws)
        pltpu.sync_copy(b_hbm, bsv)
        def fire_loads(p):
            row0 = (wid * NPASS + p) * CHUNK
            return [pltpu.async_copy(x_hbm.at[row0 + b],
                                     xs.at[pl.ds(b * D, D)], sem)
                    for b in range(CHUNK)]

        stores = []
        loads = fire_loads(0)
        for p in range(NPASS):
            for cp in loads:
                cp.wait()
            _sc_layer(xs, hs, idxs, ws, bsv, *_LAYERS[0])
            _sc_layer(hs, xs, idxs, ws, bsv, *_LAYERS[1])
            _sc_layer(xs, hs, idxs, ws, bsv, *_LAYERS[2])
            row0 = (wid * NPASS + p) * CHUNK
            stores = [pltpu.async_copy(hs.at[pl.ds(b * D, D)],
                                       out_hbm.at[row0 + b], sem2)
                      for b in range(CHUNK)]
            if p + 1 < NPASS:
                loads = fire_loads(p + 1)
            for cp in stores:
                cp.wait()

    return sc_net


@functools.cache
def _sc_net_cached():
    return _make_sc_net()


def _tc_head_body(h_ref, w_ref, b_ref, o_ref):
    g = _gelu_fast(h_ref[...])
    logits = jnp.dot(g, w_ref[...], preferred_element_type=jnp.float32,
                     precision=lax.Precision.HIGHEST) + b_ref[...]
    m = jnp.max(logits, axis=-1, keepdims=True)
    e = jnp.exp(logits - m)
    o_ref[...] = e / jnp.sum(e, axis=-1, keepdims=True)


def _tc_head(h3, fcw_t, fc_b2):
    blk = 1024
    nrows = h3.shape[0]
    return pl.pallas_call(
        _tc_head_body,
        grid=(nrows // blk,),
        in_specs=[
            pl.BlockSpec((blk, D), lambda i: (i, 0)),
            pl.BlockSpec((D, NUM_CLASSES), lambda i: (0, 0)),
            pl.BlockSpec((1, NUM_CLASSES), lambda i: (0, 0)),
        ],
        out_specs=pl.BlockSpec((blk, NUM_CLASSES), lambda i: (i, 0)),
        out_shape=jax.ShapeDtypeStruct((nrows, NUM_CLASSES), jnp.float32),
    )(h3, fcw_t, fc_b2)


def _densify_t(idx_ref, w_ref, kk):
    ii = lax.broadcasted_iota(jnp.int32, (D, D), 0)
    wt = jnp.zeros((D, D), jnp.float32)
    for k in range(kk):
        idx_k = idx_ref[...][:, k][None, :]
        w_k = w_ref[...][:, k][None, :]
        wt = wt + jnp.where(ii == idx_k, w_k, 0.0)
    return wt


def _tc_chain_body(x_ref, i1, w1r, b1r, i2, w2r, b2r, i3, w3r, b3r,
                   fw_ref, fb_ref, o_ref):
    h = x_ref[...]
    for iref, wref, bref, kk, g in ((i1, w1r, b1r, 2, True),
                                    (i2, w2r, b2r, 4, True),
                                    (i3, w3r, b3r, 8, False)):
        wt = _densify_t(iref, wref, kk)
        h = jnp.dot(h, wt, preferred_element_type=jnp.float32) + bref[...]
        if g:
            h = _gelu_fast(h)
    g3 = _gelu_fast(h)
    logits = jnp.dot(g3, fw_ref[...], preferred_element_type=jnp.float32) \
        + fb_ref[...]
    m = jnp.max(logits, axis=-1, keepdims=True)
    e = jnp.exp(logits - m)
    o_ref[...] = e / jnp.sum(e, axis=-1, keepdims=True)


def _tc_chain(x_tc, sparse_params, fcw_t, fc_b2):
    return pl.pallas_call(
        _tc_chain_body,
        out_shape=jax.ShapeDtypeStruct((B_TC, NUM_CLASSES), jnp.float32),
    )(x_tc, *sparse_params, fcw_t, fc_b2)


def kernel(x, idx1, w1, b1, idx2, w2, b2, idx3, w3, b3, fc_w, fc_b):
    idx_all = jnp.concatenate(
        [idx1.T.astype(jnp.int32), idx2.T.astype(jnp.int32),
         idx3.T.astype(jnp.int32)], axis=0).reshape(-1)
    w_all = jnp.concatenate([w1.T, w2.T, w3.T], axis=0).reshape(-1)
    b_all = jnp.concatenate([b1, b2, b3], axis=0)

    fcw_t = fc_w.T
    fc_b2 = fc_b.reshape(1, NUM_CLASSES)

    sparse_params = (
        idx1.astype(jnp.int32), w1, b1.reshape(1, D),
        idx2.astype(jnp.int32), w2, b2.reshape(1, D),
        idx3.astype(jnp.int32), w3, b3.reshape(1, D),
    )
    probs_tc = _tc_chain(x[B_SC:], sparse_params, fcw_t, fc_b2)
    h3 = _sc_net_cached()(x[:B_SC], idx_all, w_all, b_all)
    probs_sc = _tc_head(h3, fcw_t, fc_b2)
    return jnp.concatenate([probs_sc, probs_tc], axis=0)


# speedup vs baseline: 2.6028x; 1.0863x over previous
import functools

import jax
import jax.numpy as jnp
from jax import lax
from jax.experimental import pallas as pl
from jax.experimental.pallas import tpu as pltpu
from jax.experimental.pallas import tpu_sc as plsc

B = 4096
D = 784
NUM_CLASSES = 10

B_SC = 1024
B_TC = B - B_SC

NC = 2
NS = 16
NW = NC * NS
CHUNK = 32
NPASS = B_SC // (NW * CHUNK)
OCN = D // 16

_LAYERS = ((0, 2, 0, True), (2, 4, 1, True), (6, 8, 2, False))
_KTOT = 14


def _gelu_exact(v):
    z = v * 0.7071067811865476
    a = jnp.abs(z)
    t = 1.0 / (1.0 + 0.3275911 * a)
    poly = t * (0.254829592 + t * (-0.284496736 + t * (1.421413741
             + t * (-1.453152027 + t * 1.061405429))))
    erf_a = 1.0 - poly * jnp.exp(-(a * a))
    erf_z = jnp.where(z < 0.0, -erf_a, erf_a)
    return 0.5 * v * (1.0 + erf_z)


def _gelu_fast(v):
    t = jnp.minimum(v * v, 90.0)
    u = -0.0007098086084286619 * t + 0.07405305138626019
    u = u * t + 1.5949698227920912
    e = jnp.exp(-(u * v))
    return v / (1.0 + e)


def _sc_layer(src, dst, idxs, ws, bsv, k0, kk, brow, do_gelu):
    @plsc.parallel_loop(0, OCN)
    def oc_body(oc):
        col = oc * 16
        bias = bsv[pl.ds(brow * D + col, 16)]
        taps = [(idxs[pl.ds((k0 + k) * D + col, 16)],
                 ws[pl.ds((k0 + k) * D + col, 16)]) for k in range(kk)]

        @plsc.parallel_loop(0, CHUNK, unroll=4)
        def b_body(b):
            boff = b * D
            acc = bias
            for rvec, wvec in taps:
                vals = plsc.load_gather(src, [rvec + boff])
                acc = acc + vals * wvec
            if do_gelu:
                acc = _gelu_fast(acc)
            dst[pl.ds(boff + col, 16)] = acc


def _make_sc_net():
    mesh = plsc.VectorSubcoreMesh(core_axis_name="c", subcore_axis_name="s",
                                  num_cores=NC, num_subcores=NS)

    @functools.partial(
        pl.kernel,
        out_type=jax.ShapeDtypeStruct((B_SC, D), jnp.float32),
        mesh=mesh,
        compiler_params=pltpu.CompilerParams(
            needs_layout_passes=False, use_tc_tiling_on_sc=False),
        scratch_types=[
            pltpu.VMEM((CHUNK * D,), jnp.float32),
            pltpu.VMEM((CHUNK * D,), jnp.float32),
            pltpu.VMEM((_KTOT * D,), jnp.int32),
            pltpu.VMEM((_KTOT * D,), jnp.float32),
            pltpu.VMEM((3 * D,), jnp.float32),
            pltpu.SemaphoreType.DMA,
            pltpu.SemaphoreType.DMA,
        ],
    )
    def sc_net(x_hbm, idx_hbm, w_hbm, b_hbm, out_hbm,
               xs, hs, idxs, ws, bsv, sem, sem2):
        wid = lax.axis_index("s") * NC + lax.axis_index("c")
        pltpu.sync_copy(idx_hbm, idxs)
        pltpu.sync_copy(w_hbm, ws)
        pltpu.sync_copy(b_hbm, bsv)
        def fire_loads(p):
            row0 = (wid * NPASS + p) * CHUNK
            return [pltpu.async_copy(x_hbm.at[row0 + b],
                                     xs.at[pl.ds(b * D, D)], sem)
                    for b in range(CHUNK)]

        stores = []
        loads = fire_loads(0)
        for p in range(NPASS):
            for cp in loads:
                cp.wait()
            _sc_layer(xs, hs, idxs, ws, bsv, *_LAYERS[0])
            _sc_layer(hs, xs, idxs, ws, bsv, *_LAYERS[1])
            _sc_layer(xs, hs, idxs, ws, bsv, *_LAYERS[2])
            row0 = (wid * NPASS + p) * CHUNK
            stores = [pltpu.async_copy(hs.at[pl.ds(b * D, D)],
                                       out_hbm.at[row0 + b], sem2)
                      for b in range(CHUNK)]
            if p + 1 < NPASS:
                loads = fire_loads(p + 1)
            for cp in stores:
                cp.wait()

    return sc_net


@functools.cache
def _sc_net_cached():
    return _make_sc_net()


def _tc_head_body(h_ref, w_ref, b_ref, o_ref):
    g = _gelu_fast(h_ref[...])
    logits = jnp.dot(g, w_ref[...], preferred_element_type=jnp.float32,
                     precision=lax.Precision.HIGHEST) + b_ref[...]
    m = jnp.max(logits, axis=-1, keepdims=True)
    e = jnp.exp(logits - m)
    o_ref[...] = e / jnp.sum(e, axis=-1, keepdims=True)


def _tc_head(h3, fcw_t, fc_b2):
    blk = 1024
    nrows = h3.shape[0]
    return pl.pallas_call(
        _tc_head_body,
        grid=(nrows // blk,),
        in_specs=[
            pl.BlockSpec((blk, D), lambda i: (i, 0)),
            pl.BlockSpec((D, NUM_CLASSES), lambda i: (0, 0)),
            pl.BlockSpec((1, NUM_CLASSES), lambda i: (0, 0)),
        ],
        out_specs=pl.BlockSpec((blk, NUM_CLASSES), lambda i: (i, 0)),
        out_shape=jax.ShapeDtypeStruct((nrows, NUM_CLASSES), jnp.float32),
    )(h3, fcw_t, fc_b2)


def _densify_t(idx_ref, w_ref, kk):
    ii = lax.broadcasted_iota(jnp.int32, (D, D), 0)
    wt = jnp.zeros((D, D), jnp.float32)
    for k in range(kk):
        idx_k = idx_ref[...][:, k][None, :]
        w_k = w_ref[...][:, k][None, :]
        wt = wt + jnp.where(ii == idx_k, w_k, 0.0)
    return wt


def _tc_chain_body(x_ref, i1, w1r, b1r, i2, w2r, b2r, i3, w3r, b3r,
                   fw_ref, fb_ref, o_ref):
    h = x_ref[pl.ds(B_SC, B_TC), :]
    for iref, wref, bref, kk, g in ((i1, w1r, b1r, 2, True),
                                    (i2, w2r, b2r, 4, True),
                                    (i3, w3r, b3r, 8, False)):
        wt = _densify_t(iref, wref, kk)
        h = jnp.dot(h, wt, preferred_element_type=jnp.float32) + bref[...]
        if g:
            h = _gelu_fast(h)
    g3 = _gelu_fast(h)
    logits = jnp.dot(g3, fw_ref[...], preferred_element_type=jnp.float32) \
        + fb_ref[...]
    m = jnp.max(logits, axis=-1, keepdims=True)
    e = jnp.exp(logits - m)
    o_ref[...] = e / jnp.sum(e, axis=-1, keepdims=True)


def _tc_chain(x_tc, sparse_params, fcw_t, fc_b2):
    return pl.pallas_call(
        _tc_chain_body,
        out_shape=jax.ShapeDtypeStruct((B_TC, NUM_CLASSES), jnp.float32),
    )(x_tc, *sparse_params, fcw_t, fc_b2)


def kernel(x, idx1, w1, b1, idx2, w2, b2, idx3, w3, b3, fc_w, fc_b):
    idx_all = jnp.concatenate(
        [idx1.T.astype(jnp.int32), idx2.T.astype(jnp.int32),
         idx3.T.astype(jnp.int32)], axis=0).reshape(-1)
    w_all = jnp.concatenate([w1.T, w2.T, w3.T], axis=0).reshape(-1)
    b_all = jnp.concatenate([b1, b2, b3], axis=0)

    fcw_t = fc_w.T
    fc_b2 = fc_b.reshape(1, NUM_CLASSES)

    sparse_params = (
        idx1.astype(jnp.int32), w1, b1.reshape(1, D),
        idx2.astype(jnp.int32), w2, b2.reshape(1, D),
        idx3.astype(jnp.int32), w3, b3.reshape(1, D),
    )
    probs_tc = _tc_chain(x, sparse_params, fcw_t, fc_b2)
    h3 = _sc_net_cached()(x[:B_SC], idx_all, w_all, b_all)
    probs_sc = _tc_head(h3, fcw_t, fc_b2)
    return jnp.concatenate([probs_sc, probs_tc], axis=0)


# speedup vs baseline: 2.6070x; 1.0016x over previous
import functools

import jax
import jax.numpy as jnp
from jax import lax
from jax.experimental import pallas as pl
from jax.experimental.pallas import tpu as pltpu
from jax.experimental.pallas import tpu_sc as plsc

B = 4096
D = 784
NUM_CLASSES = 10

B_SC = 1024
B_TC = B - B_SC

NC = 2
NS = 16
NW = NC * NS
CHUNK = 32
NPASS = B_SC // (NW * CHUNK)
OCN = D // 16

_LAYERS = ((0, 2, 0, True), (2, 4, 1, True), (6, 8, 2, False))
_KTOT = 14


def _gelu_fast(v):
    t = jnp.minimum(v * v, 90.0)
    u = -0.0007098086084286619 * t + 0.07405305138626019
    u = u * t + 1.5949698227920912
    e = jnp.exp(-(u * v))
    return v / (1.0 + e)


def _sc_layer(src, dst, idxs, ws, bsv, k0, kk, brow, do_gelu):
    @plsc.parallel_loop(0, OCN)
    def oc_body(oc):
        col = oc * 16
        bias = bsv[pl.ds(brow * D + col, 16)]
        taps = [(idxs[pl.ds((k0 + k) * D + col, 16)],
                 ws[pl.ds((k0 + k) * D + col, 16)]) for k in range(kk)]

        @plsc.parallel_loop(0, CHUNK, unroll=4)
        def b_body(b):
            boff = b * D
            acc = bias
            for rvec, wvec in taps:
                vals = plsc.load_gather(src, [rvec + boff])
                acc = acc + vals * wvec
            if do_gelu:
                acc = _gelu_fast(acc)
            dst[pl.ds(boff + col, 16)] = acc


def _make_sc_net():
    mesh = plsc.VectorSubcoreMesh(core_axis_name="c", subcore_axis_name="s",
                                  num_cores=NC, num_subcores=NS)

    @functools.partial(
        pl.kernel,
        out_type=jax.ShapeDtypeStruct((B_SC, D), jnp.float32),
        mesh=mesh,
        compiler_params=pltpu.CompilerParams(
            needs_layout_passes=False, use_tc_tiling_on_sc=False),
        scratch_types=[
            pltpu.VMEM((CHUNK * D,), jnp.float32),
            pltpu.VMEM((CHUNK * D,), jnp.float32),
            pltpu.VMEM((_KTOT * D,), jnp.int32),
            pltpu.VMEM((_KTOT * D,), jnp.float32),
            pltpu.VMEM((3 * D,), jnp.float32),
            pltpu.SemaphoreType.DMA,
            pltpu.SemaphoreType.DMA,
        ],
    )
    def sc_net(x_hbm, idx_hbm, w_hbm, b_hbm, out_hbm,
               xs, hs, idxs, ws, bsv, sem, sem2):
        wid = lax.axis_index("s") * NC + lax.axis_index("c")
        pltpu.sync_copy(idx_hbm, idxs)
        pltpu.sync_copy(w_hbm, ws)
        pltpu.sync_copy(b_hbm, bsv)
        def fire_loads(p):
            row0 = (wid * NPASS + p) * CHUNK
            return [pltpu.async_copy(x_hbm.at[row0 + b],
                                     xs.at[pl.ds(b * D, D)], sem)
                    for b in range(CHUNK)]

        stores = []
        loads = fire_loads(0)
        for p in range(NPASS):
            for cp in loads:
                cp.wait()
            _sc_layer(xs, hs, idxs, ws, bsv, *_LAYERS[0])
            _sc_layer(hs, xs, idxs, ws, bsv, *_LAYERS[1])
            _sc_layer(xs, hs, idxs, ws, bsv, *_LAYERS[2])
            row0 = (wid * NPASS + p) * CHUNK
            stores = [pltpu.async_copy(hs.at[pl.ds(b * D, D)],
                                       out_hbm.at[row0 + b], sem2)
                      for b in range(CHUNK)]
            if p + 1 < NPASS:
                loads = fire_loads(p + 1)
            for cp in stores:
                cp.wait()

    return sc_net


@functools.cache
def _sc_net_cached():
    return _make_sc_net()


def _tc_head_body(h_ref, w_ref, b_ref, o_ref):
    g = _gelu_fast(h_ref[...])
    logits = jnp.dot(g, w_ref[...], preferred_element_type=jnp.float32,
                     precision=lax.Precision.HIGHEST) + b_ref[...]
    m = jnp.max(logits, axis=-1, keepdims=True)
    e = jnp.exp(logits - m)
    o_ref[...] = e / jnp.sum(e, axis=-1, keepdims=True)


def _tc_head(h3, fcw_t, fc_b2):
    blk = 1024
    nrows = h3.shape[0]
    return pl.pallas_call(
        _tc_head_body,
        grid=(nrows // blk,),
        in_specs=[
            pl.BlockSpec((blk, D), lambda i: (i, 0)),
            pl.BlockSpec((D, NUM_CLASSES), lambda i: (0, 0)),
            pl.BlockSpec((1, NUM_CLASSES), lambda i: (0, 0)),
        ],
        out_specs=pl.BlockSpec((blk, NUM_CLASSES), lambda i: (i, 0)),
        out_shape=jax.ShapeDtypeStruct((nrows, NUM_CLASSES), jnp.float32),
    )(h3, fcw_t, fc_b2)


def _densify_t(idx_ref, w_ref, kk):
    ii = lax.broadcasted_iota(jnp.int32, (D, D), 0)
    wt = jnp.zeros((D, D), jnp.float32)
    for k in range(kk):
        idx_k = idx_ref[...][:, k][None, :]
        w_k = w_ref[...][:, k][None, :]
        wt = wt + jnp.where(ii == idx_k, w_k, 0.0)
    return wt


def _tc_chain_body(x_ref, i1, w1r, b1r, i2, w2r, b2r, i3, w3r, b3r,
                   fw_ref, fb_ref, o_ref):
    h = x_ref[pl.ds(B_SC, B_TC), :]
    for iref, wref, bref, kk, g in ((i1, w1r, b1r, 2, True),
                                    (i2, w2r, b2r, 4, True),
                                    (i3, w3r, b3r, 8, False)):
        wt = _densify_t(iref, wref, kk)
        h = jnp.dot(h, wt, preferred_element_type=jnp.float32) + bref[...]
        if g:
            h = _gelu_fast(h)
    g3 = _gelu_fast(h)
    logits = jnp.dot(g3, fw_ref[...], preferred_element_type=jnp.float32) \
        + fb_ref[...]
    m = jnp.max(logits, axis=-1, keepdims=True)
    e = jnp.exp(logits - m)
    o_ref[...] = e / jnp.sum(e, axis=-1, keepdims=True)


def _tc_chain(x_tc, sparse_params, fcw_t, fc_b2):
    return pl.pallas_call(
        _tc_chain_body,
        out_shape=jax.ShapeDtypeStruct((B_TC, NUM_CLASSES), jnp.float32),
    )(x_tc, *sparse_params, fcw_t, fc_b2)


def kernel(x, idx1, w1, b1, idx2, w2, b2, idx3, w3, b3, fc_w, fc_b):
    idx_all = jnp.concatenate(
        [idx1.T.astype(jnp.int32), idx2.T.astype(jnp.int32),
         idx3.T.astype(jnp.int32)], axis=0).reshape(-1)
    w_all = jnp.concatenate([w1.T, w2.T, w3.T], axis=0).reshape(-1)
    b_all = jnp.concatenate([b1, b2, b3], axis=0)

    fcw_t = fc_w.T
    fc_b2 = fc_b.reshape(1, NUM_CLASSES)

    sparse_params = (
        idx1.astype(jnp.int32), w1, b1.reshape(1, D),
        idx2.astype(jnp.int32), w2, b2.reshape(1, D),
        idx3.astype(jnp.int32), w3, b3.reshape(1, D),
    )
    probs_tc = _tc_chain(x, sparse_params, fcw_t, fc_b2)
    h3 = _sc_net_cached()(x[:B_SC], idx_all, w_all, b_all)
    probs_sc = _tc_head(h3, fcw_t, fc_b2)
    return jnp.concatenate([probs_sc, probs_tc], axis=0)
